# baseline (speedup 1.0000x reference)
import jax
import jax.numpy as jnp
from jax.experimental import pallas as pl
from jax.experimental.pallas import tpu as pltpu

N_MOL, N_PRO, B = 1024, 4096, 16
N = N_MOL + N_PRO
CL, CP, CI = 5.0, 1.2, 1.5
NORM_FACTOR = 100.0
T = 256
NT = N // T
MOL_TILES = N_MOL // T
CH = 64
NCH = T // CH
F32 = jnp.float32

_INTERPRET = False


def _mm(a, b):
    return jax.lax.dot_general(a, b, (((a.ndim - 1,), (0,)), ((), ())),
                               preferred_element_type=F32)


def _silu(v):
    return v * jax.nn.sigmoid(v)


def _embed_kernel(f_ref, g_ref, t_ref, W1_ref, b1_ref, W2_ref, b2_ref,
                  eW_ref, eb_ref, out_ref):
    f = f_ref[...]
    henc = _mm(_silu(_mm(f, W1_ref[...]) + b1_ref[...]), W2_ref[...]) + b2_ref[...]
    iota = jax.lax.broadcasted_iota(jnp.int32, (f.shape[0], B), 1).astype(F32)
    oh = jnp.where(g_ref[...] == iota, 1.0, 0.0)
    tg = _mm(oh, t_ref[...])
    out_ref[...] = (_mm(henc, eW_ref[0:32, :]) + tg * eW_ref[32:33, :]
                    + eb_ref[...])


def _embed(f, g2d, t, W1, b1, W2, b2, eW, eb):
    m = f.shape[0]
    return pl.pallas_call(
        _embed_kernel,
        out_shape=jax.ShapeDtypeStruct((m, 64), F32),
        interpret=_INTERPRET,
    )(f, g2d, t, W1, b1, W2, b2, eW, eb)


def _rep(v, w):
    return jnp.broadcast_to(v[:, None, :], (CH, T, w)).reshape(CH * T, w)


def _tile(v, w):
    return jnp.broadcast_to(v[None, :, :], (CH, T, w)).reshape(CH * T, w)


def _mlp_inner(rt, ct, WP_ref, hr_ref, hc_ref, xr_ref, xc_ref,
               x0r_ref, x0c_ref, gr_ref, gc_ref, body):
    is_mr = rt < MOL_TILES
    is_mc = ct < MOL_TILES
    cutoff = jnp.where(jnp.logical_and(is_mr, is_mc), CL,
                       jnp.where(jnp.logical_or(is_mr, is_mc), CI, CP))
    wd = WP_ref[128:129, :]
    w0 = WP_ref[129:130, :]
    W2 = WP_ref[131:195, :]
    b2 = WP_ref[195:196, :]
    W1r = WP_ref[0:64, :]
    Bc = _mm(hc_ref[...], WP_ref[64:128, :]) + WP_ref[130:131, :]
    x0c = x0c_ref[...]
    n0c = jnp.sum(x0c * x0c, axis=1, keepdims=True)
    Bc_f = _tile(Bc, 64)
    Xc_f = _tile(xc_ref[...], 3)
    X0c_f = _tile(x0c, 3)
    gc_f = _tile(gc_ref[...], 1)
    n0c_f = _tile(n0c, 1)

    def chunk(k, carry):
        o = k * CH
        x0r = x0r_ref[pl.ds(o, CH), :]
        Xr_f = _rep(xr_ref[pl.ds(o, CH), :], 3)
        X0r_f = _rep(x0r, 3)
        n0r = jnp.sum(x0r * x0r, axis=1, keepdims=True)
        cross0 = jnp.sum(X0r_f * X0c_f, axis=1, keepdims=True)
        d2m = _rep(n0r, 1) + n0c_f - 2.0 * cross0
        distm = jnp.sqrt(jnp.maximum(d2m, 0.0))
        mask = jnp.where(
            jnp.logical_and(_rep(gr_ref[pl.ds(o, CH), :], 1) == gc_f,
                            distm <= cutoff),
            1.0, 0.0)
        dx0 = X0r_f - X0c_f
        dist0 = jnp.sum(dx0 * dx0, axis=1, keepdims=True)
        dxf = Xr_f - Xc_f
        d_l = jnp.sum(dxf * dxf, axis=1, keepdims=True)
        Ar = _mm(hr_ref[pl.ds(o, CH), :], W1r)
        t1 = _rep(Ar, 64) + Bc_f + d_l * wd + dist0 * w0
        u = _silu(t1)
        s2 = _silu(_mm(u, W2) + b2)
        body(o, s2, mask, dxf, d_l)
        return carry

    jax.lax.fori_loop(0, NCH, chunk, 0)


def _edge_kernel(glo_ref, ghi_ref, hr_ref, hc_ref, xr_ref, xc_ref,
                 x0r_ref, x0c_ref, gr_ref, gc_ref, WP_ref, NP_ref,
                 hout_ref, agg_ref):
    rt = pl.program_id(0)
    ct = pl.program_id(1)

    @pl.when(ct == 0)
    def _():
        agg_ref[...] = jnp.zeros_like(agg_ref)

    active = jnp.logical_and(glo_ref[rt] <= ghi_ref[ct],
                             glo_ref[ct] <= ghi_ref[rt])

    @pl.when(active)
    def _():
        def body(o, s2, mask, dxf, d_l):
            me = s2 * mask
            agg_ref[pl.ds(o, CH), :] += me.reshape(CH, T, 64).sum(axis=1)

        _mlp_inner(rt, ct, WP_ref, hr_ref, hc_ref, xr_ref, xc_ref,
                   x0r_ref, x0c_ref, gr_ref, gc_ref, body)

    @pl.when(ct == NT - 1)
    def _():
        h = hr_ref[...]
        a = agg_ref[...] * (1.0 / NORM_FACTOR)
        hid = _silu(_mm(jnp.concatenate([h, a], axis=1), NP_ref[0:128, :])
                    + NP_ref[128:129, :])
        hout_ref[...] = h + _mm(hid, NP_ref[129:193, :]) + NP_ref[193:194, :]


def _coord_kernel(glo_ref, ghi_ref, hr_ref, hc_ref, xr_ref, xc_ref,
                  x0r_ref, x0c_ref, gr_ref, gc_ref, WP_ref,
                  xout_ref, aggx_ref):
    rt = pl.program_id(0)
    ct = pl.program_id(1)

    @pl.when(ct == 0)
    def _():
        aggx_ref[...] = jnp.zeros_like(aggx_ref)

    active = jnp.logical_and(glo_ref[rt] <= ghi_ref[ct],
                             glo_ref[ct] <= ghi_ref[rt])

    @pl.when(active)
    def _():
        w3 = WP_ref[196:197, :]

        def body(o, s2, mask, dxf, d_l):
            inv = 1.0 / (jnp.sqrt(d_l + 1e-8) + 1.0)
            phi = jnp.sum(s2 * w3, axis=1, keepdims=True) * mask * inv
            tr = dxf * phi
            aggx_ref[pl.ds(o, CH), :] += tr.reshape(CH, T, 3).sum(axis=1)

        _mlp_inner(rt, ct, WP_ref, hr_ref, hc_ref, xr_ref, xc_ref,
                   x0r_ref, x0c_ref, gr_ref, gc_ref, body)

    @pl.when(ct == NT - 1)
    def _():
        upd = jnp.where(rt < MOL_TILES, 1.0, 0.0)
        xout_ref[...] = xr_ref[...] + aggx_ref[...] * (upd / NORM_FACTOR)


def _sweep(body, extra_weight_specs, out_block, out_shape, scratch_shape):
    grid_spec = pltpu.PrefetchScalarGridSpec(
        num_scalar_prefetch=2,
        grid=(NT, NT),
        in_specs=[
            pl.BlockSpec((T, 64), lambda r, c, *_: (r, 0)),
            pl.BlockSpec((T, 64), lambda r, c, *_: (c, 0)),
            pl.BlockSpec((T, 3), lambda r, c, *_: (r, 0)),
            pl.BlockSpec((T, 3), lambda r, c, *_: (c, 0)),
            pl.BlockSpec((T, 3), lambda r, c, *_: (r, 0)),
            pl.BlockSpec((T, 3), lambda r, c, *_: (c, 0)),
            pl.BlockSpec((T, 1), lambda r, c, *_: (r, 0)),
            pl.BlockSpec((T, 1), lambda r, c, *_: (c, 0)),
        ] + extra_weight_specs,
        out_specs=[pl.BlockSpec(out_block, lambda r, c, *_: (r, 0))],
        scratch_shapes=[pltpu.VMEM(scratch_shape, F32)],
    )
    return pl.pallas_call(
        body,
        grid_spec=grid_spec,
        out_shape=[jax.ShapeDtypeStruct(out_shape, F32)],
        interpret=_INTERPRET,
    )


_W_SPEC = pl.BlockSpec((200, 64), lambda r, c, *_: (0, 0))


def _edge_sweep(glo, ghi, h, x, x0, g2d, WP, NP):
    call = _sweep(_edge_kernel, [_W_SPEC, _W_SPEC], (T, 64), (N, 64), (T, 64))
    return call(glo, ghi, h, h, x, x, x0, x0, g2d, g2d, WP, NP)[0]


def _coord_sweep(glo, ghi, h, x, x0, g2d, WP):
    call = _sweep(_coord_kernel, [_W_SPEC], (T, 3), (N, 3), (T, 3))
    return call(glo, ghi, h, h, x, x, x0, x0, g2d, g2d, WP)[0]


def _out_kernel(h_ref, x_ref, x0_ref, oW_ref, ob_ref, dW1_ref, db1_ref,
                dW2_ref, db2_ref, o_ref):
    ho = _mm(h_ref[...], oW_ref[...]) + ob_ref[...]
    h32 = ho[:, 0:32]
    dec = (_mm(_silu(_mm(h32, dW1_ref[...]) + db1_ref[...]), dW2_ref[...])
           + db2_ref[...])
    vel = x_ref[...] - x0_ref[...]
    o_ref[...] = jnp.concatenate([vel, dec], axis=1)


def _decode(h, x, x0, oW, ob, dW1, db1, dW2, db2, width):
    m = h.shape[0]
    return pl.pallas_call(
        _out_kernel,
        out_shape=jax.ShapeDtypeStruct((m, 3 + width), F32),
        interpret=_INTERPRET,
    )(h, x, x0, oW, ob, dW1, db1, dW2, db2)


def _pack_sweep_weights(W1, b1, W2, b2, w3):
    rows = [
        W1[0:64],
        W1[64:128],
        W1[128:129],
        W1[129:130],
        b1[None, :],
        W2,
        b2[None, :],
        w3[None, :],
        jnp.zeros((3, 64), F32),
    ]
    return jnp.concatenate(rows, axis=0)


def _pack_node_weights(p, l):
    rows = [
        p["node_W1"][l],
        p["node_b1"][l][None, :],
        p["node_W2"][l],
        p["node_b2"][l][None, :],
        jnp.zeros((6, 64), F32),
    ]
    return jnp.concatenate(rows, axis=0)


def kernel(z_t_mol, z_t_pro, t, molecule_idx, protein_pocket_idx, params):
    p = params
    x_mol, x_pro = z_t_mol[:, :3], z_t_pro[:, :3]
    f_mol, f_pro = z_t_mol[:, 3:], z_t_pro[:, 3:]
    x0 = jnp.concatenate([x_mol, x_pro], axis=0)
    gid = jnp.concatenate([molecule_idx, protein_pocket_idx]).astype(F32)
    g2d = gid[:, None]
    glo = jnp.concatenate([molecule_idx.reshape(MOL_TILES, T)[:, 0],
                           protein_pocket_idx.reshape(NT - MOL_TILES, T)[:, 0]])
    ghi = jnp.concatenate([molecule_idx.reshape(MOL_TILES, T)[:, -1],
                           protein_pocket_idx.reshape(NT - MOL_TILES, T)[:, -1]])

    r1 = lambda v: v[None, :]
    h_mol = _embed(f_mol, g2d[:N_MOL], t, p["ae_W1"], r1(p["ae_b1"]),
                   p["ae_W2"], r1(p["ae_b2"]), p["emb_W"], r1(p["emb_b"]))
    h_pro = _embed(f_pro, g2d[N_MOL:], t, p["re_W1"], r1(p["re_b1"]),
                   p["re_W2"], r1(p["re_b2"]), p["emb_W"], r1(p["emb_b"]))
    h = jnp.concatenate([h_mol, h_pro], axis=0)

    x = x0
    zero64 = jnp.zeros((64,), F32)
    for l in range(2):
        WE = _pack_sweep_weights(p["edge_W1"][l], p["edge_b1"][l],
                                 p["edge_W2"][l], p["edge_b2"][l], zero64)
        WC = _pack_sweep_weights(p["coord_W1"][l], p["coord_b1"][l],
                                 p["coord_W2"][l], p["coord_b2"][l],
                                 p["coord_W3"][l][:, 0])
        NP = _pack_node_weights(p, l)
        h = _edge_sweep(glo, ghi, h, x, x0, g2d, WE, NP)
        x = _coord_sweep(glo, ghi, h, x, x0, g2d, WC)

    eps_mol = _decode(h[:N_MOL], x[:N_MOL], x0[:N_MOL],
                      p["out_W"], r1(p["out_b"]), p["ad_W1"], r1(p["ad_b1"]),
                      p["ad_W2"], r1(p["ad_b2"]), 16)
    eps_pro = _decode(h[N_MOL:], x[N_MOL:], x0[N_MOL:],
                      p["out_W"], r1(p["out_b"]), p["rd_W1"], r1(p["rd_b1"]),
                      p["rd_W2"], r1(p["rd_b2"]), 20)
    return eps_mol, eps_pro


# speedup vs baseline: 1.4255x; 1.4255x over previous
import jax
import jax.numpy as jnp
from jax.experimental import pallas as pl
from jax.experimental.pallas import tpu as pltpu

N_MOL, N_PRO, B = 1024, 4096, 16
N = N_MOL + N_PRO
CL, CP, CI = 5.0, 1.2, 1.5
NORM_FACTOR = 100.0
T = 256
NT = N // T
MOL_TILES = N_MOL // T
CH = 64
NCH = T // CH
F32 = jnp.float32

_INTERPRET = False


BF = jnp.bfloat16


def _mm(a, b, out=None):
    return jax.lax.dot_general(a, b, (((a.ndim - 1,), (0,)), ((), ())),
                               preferred_element_type=out or F32)


def _silu(v):
    return v * jax.nn.sigmoid(v)


def _embed_kernel(f_ref, g_ref, t_ref, W1_ref, b1_ref, W2_ref, b2_ref,
                  eW_ref, eb_ref, out_ref):
    f = f_ref[...]
    henc = _mm(_silu(_mm(f, W1_ref[...]) + b1_ref[...]), W2_ref[...]) + b2_ref[...]
    iota = jax.lax.broadcasted_iota(jnp.int32, (f.shape[0], B), 1).astype(F32)
    oh = jnp.where(g_ref[...] == iota, 1.0, 0.0)
    tg = _mm(oh, t_ref[...])
    out_ref[...] = (_mm(henc, eW_ref[0:32, :]) + tg * eW_ref[32:33, :]
                    + eb_ref[...])


def _embed(f, g2d, t, W1, b1, W2, b2, eW, eb):
    m = f.shape[0]
    return pl.pallas_call(
        _embed_kernel,
        out_shape=jax.ShapeDtypeStruct((m, 64), F32),
        interpret=_INTERPRET,
    )(f, g2d, t, W1, b1, W2, b2, eW, eb)


def _rep(v, w):
    return jnp.broadcast_to(v[:, None, :], (CH, T, w)).reshape(CH * T, w)


def _tile(v, w):
    return jnp.broadcast_to(v[None, :, :], (CH, T, w)).reshape(CH * T, w)


def _mlp_inner(rt, ct, clo, chi, c64lo_ref, c64hi_ref,
               WP_ref, hr_ref, hc_ref, xr_ref, xc_ref,
               x0r_ref, x0c_ref, gr_ref, gc_ref, body):
    is_mr = rt < MOL_TILES
    is_mc = ct < MOL_TILES
    cutoff = jnp.where(jnp.logical_and(is_mr, is_mc), CL,
                       jnp.where(jnp.logical_or(is_mr, is_mc), CI, CP))
    WD = WP_ref[128:130, :].astype(BF)
    W2 = WP_ref[131:195, :].astype(BF)
    b2 = WP_ref[195:196, :]
    W1r = WP_ref[0:64, :]
    Bc = _mm(hc_ref[...], WP_ref[64:128, :]) + WP_ref[130:131, :]
    x0c = x0c_ref[...]
    n0c = jnp.sum(x0c * x0c, axis=1, keepdims=True)
    Bc_f = _tile(Bc.astype(BF), 64)
    Xc_f = _tile(xc_ref[...], 3)
    X0c_f = _tile(x0c, 3)
    gc_f = _tile(gc_ref[...], 1)
    n0c_f = _tile(n0c, 1)

    def chunk(k, carry):
        o = k * CH
        x0r = x0r_ref[pl.ds(o, CH), :]
        Xr_f = _rep(xr_ref[pl.ds(o, CH), :], 3)
        X0r_f = _rep(x0r, 3)
        n0r = jnp.sum(x0r * x0r, axis=1, keepdims=True)
        cross0 = jnp.sum(X0r_f * X0c_f, axis=1, keepdims=True)
        d2m = _rep(n0r, 1) + n0c_f - 2.0 * cross0
        distm = jnp.sqrt(jnp.maximum(d2m, 0.0))
        mask = jnp.where(
            jnp.logical_and(_rep(gr_ref[pl.ds(o, CH), :], 1) == gc_f,
                            distm <= cutoff),
            1.0, 0.0)
        dx0 = X0r_f - X0c_f
        dist0 = jnp.sum(dx0 * dx0, axis=1, keepdims=True)
        dxf = Xr_f - Xc_f
        d_l = jnp.sum(dxf * dxf, axis=1, keepdims=True)
        Ar = _mm(hr_ref[pl.ds(o, CH), :], W1r)
        side = _mm(jnp.concatenate([d_l, dist0], axis=1).astype(BF),
                   WD).astype(BF)
        t1 = _rep(Ar.astype(BF), 64) + Bc_f + side
        u = _silu(t1)
        s2 = _silu((_mm(u, W2) + b2).astype(BF))
        body(o, s2, mask, dxf, d_l)
        return carry

    base = rt * NCH
    k0 = jnp.int32(NCH)
    k1 = jnp.int32(0)
    for k in range(NCH):
        ov = jnp.logical_and(c64lo_ref[base + k] <= chi,
                             clo <= c64hi_ref[base + k])
        k0 = jnp.where(ov, jnp.minimum(k0, k), k0)
        k1 = jnp.where(ov, k + 1, k1)
    jax.lax.fori_loop(k0, k1, chunk, 0)


def _edge_kernel(glo_ref, ghi_ref, c64lo_ref, c64hi_ref,
                 hr_ref, hc_ref, xr_ref, xc_ref,
                 x0r_ref, x0c_ref, gr_ref, gc_ref, WP_ref, NP_ref,
                 hout_ref, agg_ref):
    rt = pl.program_id(0)
    ct = pl.program_id(1)

    @pl.when(ct == 0)
    def _():
        agg_ref[...] = jnp.zeros_like(agg_ref)

    active = jnp.logical_and(glo_ref[rt] <= ghi_ref[ct],
                             glo_ref[ct] <= ghi_ref[rt])

    @pl.when(active)
    def _():
        def body(o, s2, mask, dxf, d_l):
            me = s2 * mask.astype(BF)
            agg_ref[pl.ds(o, CH), :] += me.reshape(CH, T, 64).sum(
                axis=1, dtype=F32)

        _mlp_inner(rt, ct, glo_ref[ct], ghi_ref[ct], c64lo_ref, c64hi_ref,
                   WP_ref, hr_ref, hc_ref, xr_ref, xc_ref,
                   x0r_ref, x0c_ref, gr_ref, gc_ref, body)

    @pl.when(ct == NT - 1)
    def _():
        h = hr_ref[...]
        a = agg_ref[...] * (1.0 / NORM_FACTOR)
        hid = _silu(_mm(jnp.concatenate([h, a], axis=1), NP_ref[0:128, :])
                    + NP_ref[128:129, :])
        hout_ref[...] = h + _mm(hid, NP_ref[129:193, :]) + NP_ref[193:194, :]


def _coord_kernel(glo_ref, ghi_ref, c64lo_ref, c64hi_ref,
                  hr_ref, hc_ref, xr_ref, xc_ref,
                  x0r_ref, x0c_ref, gr_ref, gc_ref, WP_ref,
                  xout_ref, aggx_ref):
    rt = pl.program_id(0)
    ct = pl.program_id(1)

    @pl.when(ct == 0)
    def _():
        aggx_ref[...] = jnp.zeros_like(aggx_ref)

    active = jnp.logical_and(glo_ref[rt] <= ghi_ref[ct],
                             glo_ref[ct] <= ghi_ref[rt])

    @pl.when(active)
    def _():
        w3 = WP_ref[196:197, :]

        def body(o, s2, mask, dxf, d_l):
            inv = 1.0 / (jnp.sqrt(d_l + 1e-8) + 1.0)
            phi = (jnp.sum(s2 * w3.astype(BF), axis=1, keepdims=True,
                           dtype=F32) * mask * inv)
            tr = dxf * phi
            aggx_ref[pl.ds(o, CH), :] += tr.reshape(CH, T, 3).sum(axis=1)

        _mlp_inner(rt, ct, glo_ref[ct], ghi_ref[ct], c64lo_ref, c64hi_ref,
                   WP_ref, hr_ref, hc_ref, xr_ref, xc_ref,
                   x0r_ref, x0c_ref, gr_ref, gc_ref, body)

    @pl.when(ct == NT - 1)
    def _():
        upd = jnp.where(rt < MOL_TILES, 1.0, 0.0)
        xout_ref[...] = xr_ref[...] + aggx_ref[...] * (upd / NORM_FACTOR)


def _sweep(body, extra_weight_specs, out_block, out_shape, scratch_shape):
    grid_spec = pltpu.PrefetchScalarGridSpec(
        num_scalar_prefetch=4,
        grid=(NT, NT),
        in_specs=[
            pl.BlockSpec((T, 64), lambda r, c, *_: (r, 0)),
            pl.BlockSpec((T, 64), lambda r, c, *_: (c, 0)),
            pl.BlockSpec((T, 3), lambda r, c, *_: (r, 0)),
            pl.BlockSpec((T, 3), lambda r, c, *_: (c, 0)),
            pl.BlockSpec((T, 3), lambda r, c, *_: (r, 0)),
            pl.BlockSpec((T, 3), lambda r, c, *_: (c, 0)),
            pl.BlockSpec((T, 1), lambda r, c, *_: (r, 0)),
            pl.BlockSpec((T, 1), lambda r, c, *_: (c, 0)),
        ] + extra_weight_specs,
        out_specs=[pl.BlockSpec(out_block, lambda r, c, *_: (r, 0))],
        scratch_shapes=[pltpu.VMEM(scratch_shape, F32)],
    )
    return pl.pallas_call(
        body,
        grid_spec=grid_spec,
        out_shape=[jax.ShapeDtypeStruct(out_shape, F32)],
        interpret=_INTERPRET,
    )


_W_SPEC = pl.BlockSpec((200, 64), lambda r, c, *_: (0, 0))


def _edge_sweep(glo, ghi, c64lo, c64hi, h, x, x0, g2d, WP, NP):
    call = _sweep(_edge_kernel, [_W_SPEC, _W_SPEC], (T, 64), (N, 64), (T, 64))
    return call(glo, ghi, c64lo, c64hi, h, h, x, x, x0, x0, g2d, g2d,
                WP, NP)[0]


def _coord_sweep(glo, ghi, c64lo, c64hi, h, x, x0, g2d, WP):
    call = _sweep(_coord_kernel, [_W_SPEC], (T, 3), (N, 3), (T, 3))
    return call(glo, ghi, c64lo, c64hi, h, h, x, x, x0, x0, g2d, g2d, WP)[0]


def _out_kernel(h_ref, x_ref, x0_ref, oW_ref, ob_ref, dW1_ref, db1_ref,
                dW2_ref, db2_ref, o_ref):
    ho = _mm(h_ref[...], oW_ref[...]) + ob_ref[...]
    h32 = ho[:, 0:32]
    dec = (_mm(_silu(_mm(h32, dW1_ref[...]) + db1_ref[...]), dW2_ref[...])
           + db2_ref[...])
    vel = x_ref[...] - x0_ref[...]
    o_ref[...] = jnp.concatenate([vel, dec], axis=1)


def _decode(h, x, x0, oW, ob, dW1, db1, dW2, db2, width):
    m = h.shape[0]
    return pl.pallas_call(
        _out_kernel,
        out_shape=jax.ShapeDtypeStruct((m, 3 + width), F32),
        interpret=_INTERPRET,
    )(h, x, x0, oW, ob, dW1, db1, dW2, db2)


def _pack_sweep_weights(W1, b1, W2, b2, w3):
    rows = [
        W1[0:64],
        W1[64:128],
        W1[128:129],
        W1[129:130],
        b1[None, :],
        W2,
        b2[None, :],
        w3[None, :],
        jnp.zeros((3, 64), F32),
    ]
    return jnp.concatenate(rows, axis=0)


def _pack_node_weights(p, l):
    rows = [
        p["node_W1"][l],
        p["node_b1"][l][None, :],
        p["node_W2"][l],
        p["node_b2"][l][None, :],
        jnp.zeros((6, 64), F32),
    ]
    return jnp.concatenate(rows, axis=0)


def kernel(z_t_mol, z_t_pro, t, molecule_idx, protein_pocket_idx, params):
    p = params
    x_mol, x_pro = z_t_mol[:, :3], z_t_pro[:, :3]
    f_mol, f_pro = z_t_mol[:, 3:], z_t_pro[:, 3:]
    x0 = jnp.concatenate([x_mol, x_pro], axis=0)
    gid = jnp.concatenate([molecule_idx, protein_pocket_idx]).astype(F32)
    g2d = gid[:, None]
    glo = jnp.concatenate([molecule_idx.reshape(MOL_TILES, T)[:, 0],
                           protein_pocket_idx.reshape(NT - MOL_TILES, T)[:, 0]])
    ghi = jnp.concatenate([molecule_idx.reshape(MOL_TILES, T)[:, -1],
                           protein_pocket_idx.reshape(NT - MOL_TILES, T)[:, -1]])
    gid_i = jnp.concatenate([molecule_idx, protein_pocket_idx])
    c64lo = gid_i.reshape(NT * NCH, CH)[:, 0]
    c64hi = gid_i.reshape(NT * NCH, CH)[:, -1]

    r1 = lambda v: v[None, :]
    h_mol = _embed(f_mol, g2d[:N_MOL], t, p["ae_W1"], r1(p["ae_b1"]),
                   p["ae_W2"], r1(p["ae_b2"]), p["emb_W"], r1(p["emb_b"]))
    h_pro = _embed(f_pro, g2d[N_MOL:], t, p["re_W1"], r1(p["re_b1"]),
                   p["re_W2"], r1(p["re_b2"]), p["emb_W"], r1(p["emb_b"]))
    h = jnp.concatenate([h_mol, h_pro], axis=0)

    x = x0
    zero64 = jnp.zeros((64,), F32)
    for l in range(2):
        WE = _pack_sweep_weights(p["edge_W1"][l], p["edge_b1"][l],
                                 p["edge_W2"][l], p["edge_b2"][l], zero64)
        WC = _pack_sweep_weights(p["coord_W1"][l], p["coord_b1"][l],
                                 p["coord_W2"][l], p["coord_b2"][l],
                                 p["coord_W3"][l][:, 0])
        NP = _pack_node_weights(p, l)
        h = _edge_sweep(glo, ghi, c64lo, c64hi, h, x, x0, g2d, WE, NP)
        x = _coord_sweep(glo, ghi, c64lo, c64hi, h, x, x0, g2d, WC)

    eps_mol = _decode(h[:N_MOL], x[:N_MOL], x0[:N_MOL],
                      p["out_W"], r1(p["out_b"]), p["ad_W1"], r1(p["ad_b1"]),
                      p["ad_W2"], r1(p["ad_b2"]), 16)
    eps_pro = _decode(h[N_MOL:], x[N_MOL:], x0[N_MOL:],
                      p["out_W"], r1(p["out_b"]), p["rd_W1"], r1(p["rd_b1"]),
                      p["rd_W2"], r1(p["rd_b2"]), 20)
    return eps_mol, eps_pro
